# baseline (speedup 1.0000x reference)
import functools

import jax
import jax.numpy as jnp
from jax import lax
from jax.experimental import pallas as pl
from jax.experimental.pallas import tpu as pltpu
from jax.experimental.pallas import tpu_sc as plsc

N = 10000
NP = 10240
E = 320000
EP = 327680
D = 128
DH = 64
NC = 2
NS = 16
L = 16

@functools.lru_cache(maxsize=None)
def _mesh():
    return plsc.VectorSubcoreMesh(core_axis_name="c", subcore_axis_name="s",
                                  num_cores=NC, num_subcores=NS)

_GDN = lax.GatherDimensionNumbers(
    offset_dims=(), collapsed_slice_dims=(0,), start_index_map=(0,))


def _lane_bcast(vec16, j):
    idx = jnp.full((L, 1), j, jnp.int32)
    return lax.gather(vec16, idx, _GDN, slice_sizes=(1,),
                      mode=lax.GatherScatterMode.PROMISE_IN_BOUNDS)
_RPT = NP // NS

_CHD = 128
_EPTD = EP // (NC * NS)
_NCHD = _EPTD // _CHD


_BURST = 5


def _deg_body(dst_hbm, ew_hbm, pdeg_hbm, degsh, dbg, ebg, zbuf, sem):
    c = lax.axis_index("c")
    s = lax.axis_index("s")

    def _z(i, _):
        zbuf[pl.ds(i * L, L)] = jnp.zeros((L,), jnp.float32)
        return 0
    lax.fori_loop(0, _RPT // L, _z, 0)
    pltpu.sync_copy(zbuf, degsh.at[pl.ds(s * _RPT, _RPT)])

    row0 = (s * NC + c) * _NCHD
    pltpu.sync_copy(dst_hbm.at[pl.ds(row0, _NCHD), :], dbg)
    pltpu.sync_copy(ew_hbm.at[pl.ds(row0, _NCHD), :], ebg)
    plsc.subcore_barrier()

    def _burst(i, _):
        for j in range(_BURST):
            k = i * _BURST + j
            pltpu.async_copy(ebg.at[k], degsh.at[dbg.at[k]], sem, add=True)
        for j in range(_BURST):
            k = i * _BURST + j
            pltpu.make_async_copy(ebg.at[k], degsh.at[dbg.at[k]], sem).wait()
        return 0
    lax.fori_loop(0, _NCHD // _BURST, _burst, 0)

    plsc.subcore_barrier()
    pltpu.sync_copy(degsh.at[pl.ds(s * _RPT, _RPT)],
                    pdeg_hbm.at[c, pl.ds(s * _RPT, _RPT)])


@functools.lru_cache(maxsize=None)
def _get_sc_deg():
    return functools.partial(
        pl.kernel, _deg_body,
        out_type=jax.ShapeDtypeStruct((NC, NP), jnp.float32),
        mesh=_mesh(),
        scratch_types=[
            pltpu.VMEM_SHARED((NP,), jnp.float32),
            pltpu.VMEM((_NCHD, _CHD), jnp.int32),
            pltpu.VMEM((_NCHD, _CHD), jnp.float32),
            pltpu.VMEM((_RPT,), jnp.float32),
            pltpu.SemaphoreType.DMA,
        ],
        compiler_params=pltpu.CompilerParams(use_tc_tiling_on_sc=False),
    )()

_CH = 128
_EPT = EP // NS
_NCHE = _EPT // _CH
_SEG = 40


def _edge_body(hp_hbm, src_hbm, dst_hbm, ew_hbm, acc_hbm,
               table, accsh, sb, db, eb, rows0, rows1,
               gsem0, gsem1, ssem0, ssem1):
    c = lax.axis_index("c")
    s = lax.axis_index("s")
    rbase = s * _RPT

    pltpu.sync_copy(hp_hbm.at[c, pl.ds(rbase, _RPT), :],
                    table.at[pl.ds(rbase, _RPT)])
    pltpu.sync_copy(hp_hbm.at[c, pl.ds(rbase, _RPT), :],
                    accsh.at[pl.ds(rbase, _RPT)])
    plsc.subcore_barrier()

    bufs = ((rows0, gsem0, ssem0), (rows1, gsem1, ssem1))

    def _start_gather(k, b):
        pltpu.async_copy(table.at[sb.at[k]], b[0], b[1])

    def _wait_gather(k, b):
        pltpu.make_async_copy(table.at[sb.at[k]], b[0], b[1]).wait()

    def _scale(k, b):
        rows = b[0]

        def _grp(g, _):
            ew16 = eb[k, pl.ds(g * L, L)]
            for j in range(L):
                ewb = _lane_bcast(ew16, j)
                e = g * L + j
                for f in range(DH // L):
                    sl = pl.ds(f * L, L)
                    rows[e, sl] = rows[e, sl] * ewb
            return 0
        lax.fori_loop(0, _CH // L, _grp, 0)

    def _start_scatter(k, b):
        pltpu.async_copy(b[0], accsh.at[db.at[k]], b[2], add=True)

    def _wait_scatter(k, b):
        pltpu.make_async_copy(b[0], accsh.at[db.at[k]], b[2]).wait()

    def _seg(gi, _):
        segrow = s * _NCHE + gi * _SEG
        pltpu.sync_copy(src_hbm.at[pl.ds(segrow, _SEG), :], sb)
        pltpu.sync_copy(dst_hbm.at[pl.ds(segrow, _SEG), :], db)
        pltpu.sync_copy(ew_hbm.at[pl.ds(segrow, _SEG), :], eb)
        _start_gather(0, bufs[0])

        def _pair(i, _):
            k0 = 2 * i
            @pl.when(i > 0)
            def _():
                _wait_scatter(k0 - 1, bufs[1])
            _start_gather(k0 + 1, bufs[1])
            _wait_gather(k0, bufs[0])
            _scale(k0, bufs[0])
            _start_scatter(k0, bufs[0])
            @pl.when(i < _SEG // 2 - 1)
            def _():
                _wait_scatter(k0, bufs[0])
                _start_gather(k0 + 2, bufs[0])
            _wait_gather(k0 + 1, bufs[1])
            _scale(k0 + 1, bufs[1])
            _start_scatter(k0 + 1, bufs[1])
            return 0
        lax.fori_loop(0, _SEG // 2, _pair, 0)
        _wait_scatter(_SEG - 2, bufs[0])
        _wait_scatter(_SEG - 1, bufs[1])
        return 0
    lax.fori_loop(0, _NCHE // _SEG, _seg, 0)

    plsc.subcore_barrier()
    pltpu.sync_copy(accsh.at[pl.ds(rbase, _RPT)],
                    acc_hbm.at[c, pl.ds(rbase, _RPT), :])


@functools.lru_cache(maxsize=None)
def _get_sc_edge():
    return functools.partial(
        pl.kernel, _edge_body,
        out_type=jax.ShapeDtypeStruct((NC, NP, DH), jnp.float32),
        mesh=_mesh(),
        scratch_types=[
            pltpu.VMEM_SHARED((NP, DH), jnp.float32),
            pltpu.VMEM_SHARED((NP, DH), jnp.float32),
            pltpu.VMEM((_SEG, _CH), jnp.int32),
            pltpu.VMEM((_SEG, _CH), jnp.int32),
            pltpu.VMEM((_SEG, _CH), jnp.float32),
            pltpu.VMEM((_CH, DH), jnp.float32),
            pltpu.VMEM((_CH, DH), jnp.float32),
            pltpu.SemaphoreType.DMA,
            pltpu.SemaphoreType.DMA,
            pltpu.SemaphoreType.DMA,
            pltpu.SemaphoreType.DMA,
        ],
        compiler_params=pltpu.CompilerParams(use_tc_tiling_on_sc=False),
    )()

_R = 1024
_G = NP // _R


def _dinv(pdeg_ref):
    deg = pdeg_ref[:, 0:1] + pdeg_ref[:, 1:2] + 1.0
    return lax.rsqrt(deg)


def _tc_pre_body(x_ref, w_ref, pdeg_ref, o_ref):
    res = (jnp.dot(x_ref[...], w_ref[...],
                   preferred_element_type=jnp.float32) * _dinv(pdeg_ref))
    o_ref[0, :, :] = res[:, :DH]
    o_ref[1, :, :] = res[:, DH:]


def _tc_mid_body(a_ref, pdeg_ref, b_ref, w_ref, o_ref):
    dinv = _dinv(pdeg_ref)
    a = jnp.concatenate([a_ref[0], a_ref[1]], axis=-1)
    h = jnp.maximum(a * dinv + b_ref[...], 0.0)
    res = (jnp.dot(h, w_ref[...], preferred_element_type=jnp.float32) * dinv)
    o_ref[0, :, :] = res[:, :DH]
    o_ref[1, :, :] = res[:, DH:]


def _tc_post_body(a_ref, pdeg_ref, b_ref, o_ref):
    a = jnp.concatenate([a_ref[0], a_ref[1]], axis=-1)
    o_ref[...] = jnp.maximum(a * _dinv(pdeg_ref) + b_ref[...], 0.0)


_row_spec = pl.BlockSpec((_R, D), lambda i: (i, 0))
_half_spec = pl.BlockSpec((NC, _R, DH), lambda i: (0, i, 0))
_deg_spec = pl.BlockSpec((_R, 2), lambda i: (i, 0))
_w_spec = pl.BlockSpec((D, D), lambda i: (0, 0))
_b_spec = pl.BlockSpec((1, D), lambda i: (0, 0))
_half_sds = jax.ShapeDtypeStruct((NC, NP, DH), jnp.float32)

_tc_pre = pl.pallas_call(
    _tc_pre_body, grid=(_G,),
    in_specs=[_row_spec, _w_spec, _deg_spec],
    out_specs=_half_spec, out_shape=_half_sds)

_tc_mid = pl.pallas_call(
    _tc_mid_body, grid=(_G,),
    in_specs=[_half_spec, _deg_spec, _b_spec, _w_spec],
    out_specs=_half_spec, out_shape=_half_sds)

_tc_post = pl.pallas_call(
    _tc_post_body, grid=(_G,),
    in_specs=[_half_spec, _deg_spec, _b_spec],
    out_specs=_row_spec, out_shape=jax.ShapeDtypeStruct((NP, D), jnp.float32))


def kernel(x, edge_index, edge_weight, W1, b1, W2, b2):
    pad = EP - E
    src = jnp.pad(edge_index[0], (0, pad)).reshape(EP // _CH, _CH)
    dst = jnp.pad(edge_index[1], (0, pad)).reshape(EP // _CH, _CH)
    ew2 = jnp.pad(edge_weight, (0, pad)).reshape(EP // _CH, _CH)
    b1r = b1.reshape(1, D)
    b2r = b2.reshape(1, D)
    xp = jnp.pad(x, ((0, NP - N), (0, 0)))

    pdeg = _get_sc_deg()(dst, ew2)
    pdegT = pdeg.T

    sc_edge = _get_sc_edge()
    hp1 = _tc_pre(xp, W1, pdegT)
    acc1 = sc_edge(hp1, src, dst, ew2)
    hp2 = _tc_mid(acc1, pdegT, b1r, W2)
    acc2 = sc_edge(hp2, src, dst, ew2)
    return _tc_post(acc2, pdegT, b2r)[:N]


# speedup vs baseline: 1.5905x; 1.5905x over previous
import functools

import jax
import jax.numpy as jnp
from jax import lax
from jax.experimental import pallas as pl
from jax.experimental.pallas import tpu as pltpu
from jax.experimental.pallas import tpu_sc as plsc

N = 10000
NP = 10240
E = 320000
EP = 327680
D = 128
DH = 64
NC = 2
NS = 16
L = 16

@functools.lru_cache(maxsize=None)
def _mesh():
    return plsc.VectorSubcoreMesh(core_axis_name="c", subcore_axis_name="s",
                                  num_cores=NC, num_subcores=NS)

_GDN = lax.GatherDimensionNumbers(
    offset_dims=(), collapsed_slice_dims=(0,), start_index_map=(0,))


def _lane_bcast(vec16, j):
    idx = jnp.full((L, 1), j, jnp.int32)
    return lax.gather(vec16, idx, _GDN, slice_sizes=(1,),
                      mode=lax.GatherScatterMode.PROMISE_IN_BOUNDS)
_RPT = NP // NS

_CHD = 128
_EPTD = EP // (NC * NS)
_NCHD = _EPTD // _CHD


_BURST = 5


def _deg_body(dst_hbm, ew_hbm, pdeg_hbm, degsh, dbg, ebg, zbuf, sem):
    c = lax.axis_index("c")
    s = lax.axis_index("s")

    def _z(i, _):
        zbuf[pl.ds(i * L, L)] = jnp.zeros((L,), jnp.float32)
        return 0
    lax.fori_loop(0, _RPT // L, _z, 0)
    pltpu.sync_copy(zbuf, degsh.at[pl.ds(s * _RPT, _RPT)])

    row0 = (s * NC + c) * _NCHD
    pltpu.sync_copy(dst_hbm.at[pl.ds(row0, _NCHD), :], dbg)
    pltpu.sync_copy(ew_hbm.at[pl.ds(row0, _NCHD), :], ebg)
    plsc.subcore_barrier()

    def _burst(i, _):
        for j in range(_BURST):
            k = i * _BURST + j
            pltpu.async_copy(ebg.at[k], degsh.at[dbg.at[k]], sem, add=True)
        for j in range(_BURST):
            k = i * _BURST + j
            pltpu.make_async_copy(ebg.at[k], degsh.at[dbg.at[k]], sem).wait()
        return 0
    lax.fori_loop(0, _NCHD // _BURST, _burst, 0)

    plsc.subcore_barrier()
    pltpu.sync_copy(degsh.at[pl.ds(s * _RPT, _RPT)],
                    pdeg_hbm.at[c, pl.ds(s * _RPT, _RPT)])


@functools.lru_cache(maxsize=None)
def _get_sc_deg():
    return functools.partial(
        pl.kernel, _deg_body,
        out_type=jax.ShapeDtypeStruct((NC, NP), jnp.float32),
        mesh=_mesh(),
        scratch_types=[
            pltpu.VMEM_SHARED((NP,), jnp.float32),
            pltpu.VMEM((_NCHD, _CHD), jnp.int32),
            pltpu.VMEM((_NCHD, _CHD), jnp.float32),
            pltpu.VMEM((_RPT,), jnp.float32),
            pltpu.SemaphoreType.DMA,
        ],
        compiler_params=pltpu.CompilerParams(use_tc_tiling_on_sc=False),
    )()

_CH = 128
_EPT = EP // NS
_NCHE = _EPT // _CH
_SEG = 40


def _edge_body(hp_hbm, src_hbm, dst_hbm, ew_hbm, acc_hbm,
               table, accsh, sb, db, eb, rows0, rows1,
               gsem0, gsem1, ssem0, ssem1):
    c = lax.axis_index("c")
    s = lax.axis_index("s")
    rbase = s * _RPT

    pltpu.sync_copy(hp_hbm.at[c, pl.ds(rbase, _RPT), :],
                    table.at[pl.ds(rbase, _RPT)])
    pltpu.sync_copy(hp_hbm.at[c, pl.ds(rbase, _RPT), :],
                    accsh.at[pl.ds(rbase, _RPT)])
    plsc.subcore_barrier()

    bufs = ((rows0, gsem0, ssem0), (rows1, gsem1, ssem1))

    def _start_gather(k, b):
        pltpu.async_copy(table.at[sb.at[k]], b[0], b[1])

    def _wait_gather(k, b):
        pltpu.make_async_copy(table.at[sb.at[k]], b[0], b[1]).wait()

    def _scale(k, b):
        rows = b[0]
        nf = DH // L

        def _grp(g, _):
            ew16 = eb[k, pl.ds(g * L, L)]
            for j in range(0, L, 2):
                e0 = g * L + j
                e1 = e0 + 1
                ewb0 = _lane_bcast(ew16, j)
                ewb1 = _lane_bcast(ew16, j + 1)
                vals = ([rows[e0, pl.ds(f * L, L)] * ewb0 for f in range(nf)]
                        + [rows[e1, pl.ds(f * L, L)] * ewb1 for f in range(nf)])
                for f in range(nf):
                    rows[e0, pl.ds(f * L, L)] = vals[f]
                for f in range(nf):
                    rows[e1, pl.ds(f * L, L)] = vals[nf + f]
            return 0
        lax.fori_loop(0, _CH // L, _grp, 0)

    def _start_scatter(k, b):
        pltpu.async_copy(b[0], accsh.at[db.at[k]], b[2], add=True)

    def _wait_scatter(k, b):
        pltpu.make_async_copy(b[0], accsh.at[db.at[k]], b[2]).wait()

    def _seg(gi, _):
        segrow = s * _NCHE + gi * _SEG
        pltpu.sync_copy(src_hbm.at[pl.ds(segrow, _SEG), :], sb)
        pltpu.sync_copy(dst_hbm.at[pl.ds(segrow, _SEG), :], db)
        pltpu.sync_copy(ew_hbm.at[pl.ds(segrow, _SEG), :], eb)
        _start_gather(0, bufs[0])

        def _pair(i, _):
            k0 = 2 * i
            @pl.when(i > 0)
            def _():
                _wait_scatter(k0 - 1, bufs[1])
            _start_gather(k0 + 1, bufs[1])
            _wait_gather(k0, bufs[0])
            _scale(k0, bufs[0])
            _start_scatter(k0, bufs[0])
            @pl.when(i < _SEG // 2 - 1)
            def _():
                _wait_scatter(k0, bufs[0])
                _start_gather(k0 + 2, bufs[0])
            _wait_gather(k0 + 1, bufs[1])
            _scale(k0 + 1, bufs[1])
            _start_scatter(k0 + 1, bufs[1])
            return 0
        lax.fori_loop(0, _SEG // 2, _pair, 0)
        _wait_scatter(_SEG - 2, bufs[0])
        _wait_scatter(_SEG - 1, bufs[1])
        return 0
    lax.fori_loop(0, _NCHE // _SEG, _seg, 0)

    plsc.subcore_barrier()
    pltpu.sync_copy(accsh.at[pl.ds(rbase, _RPT)],
                    acc_hbm.at[c, pl.ds(rbase, _RPT), :])


@functools.lru_cache(maxsize=None)
def _get_sc_edge():
    return functools.partial(
        pl.kernel, _edge_body,
        out_type=jax.ShapeDtypeStruct((NC, NP, DH), jnp.float32),
        mesh=_mesh(),
        scratch_types=[
            pltpu.VMEM_SHARED((NP, DH), jnp.float32),
            pltpu.VMEM_SHARED((NP, DH), jnp.float32),
            pltpu.VMEM((_SEG, _CH), jnp.int32),
            pltpu.VMEM((_SEG, _CH), jnp.int32),
            pltpu.VMEM((_SEG, _CH), jnp.float32),
            pltpu.VMEM((_CH, DH), jnp.float32),
            pltpu.VMEM((_CH, DH), jnp.float32),
            pltpu.SemaphoreType.DMA,
            pltpu.SemaphoreType.DMA,
            pltpu.SemaphoreType.DMA,
            pltpu.SemaphoreType.DMA,
        ],
        compiler_params=pltpu.CompilerParams(use_tc_tiling_on_sc=False),
    )()

_R = 1024
_G = NP // _R


def _dinv(pdeg_ref):
    deg = pdeg_ref[:, 0:1] + pdeg_ref[:, 1:2] + 1.0
    return lax.rsqrt(deg)


def _tc_pre_body(x_ref, w_ref, pdeg_ref, o_ref):
    res = (jnp.dot(x_ref[...], w_ref[...],
                   preferred_element_type=jnp.float32) * _dinv(pdeg_ref))
    o_ref[0, :, :] = res[:, :DH]
    o_ref[1, :, :] = res[:, DH:]


def _tc_mid_body(a_ref, pdeg_ref, b_ref, w_ref, o_ref):
    dinv = _dinv(pdeg_ref)
    a = jnp.concatenate([a_ref[0], a_ref[1]], axis=-1)
    h = jnp.maximum(a * dinv + b_ref[...], 0.0)
    res = (jnp.dot(h, w_ref[...], preferred_element_type=jnp.float32) * dinv)
    o_ref[0, :, :] = res[:, :DH]
    o_ref[1, :, :] = res[:, DH:]


def _tc_post_body(a_ref, pdeg_ref, b_ref, o_ref):
    a = jnp.concatenate([a_ref[0], a_ref[1]], axis=-1)
    o_ref[...] = jnp.maximum(a * _dinv(pdeg_ref) + b_ref[...], 0.0)


_row_spec = pl.BlockSpec((_R, D), lambda i: (i, 0))
_half_spec = pl.BlockSpec((NC, _R, DH), lambda i: (0, i, 0))
_deg_spec = pl.BlockSpec((_R, 2), lambda i: (i, 0))
_w_spec = pl.BlockSpec((D, D), lambda i: (0, 0))
_b_spec = pl.BlockSpec((1, D), lambda i: (0, 0))
_half_sds = jax.ShapeDtypeStruct((NC, NP, DH), jnp.float32)

_tc_pre = pl.pallas_call(
    _tc_pre_body, grid=(_G,),
    in_specs=[_row_spec, _w_spec, _deg_spec],
    out_specs=_half_spec, out_shape=_half_sds)

_tc_mid = pl.pallas_call(
    _tc_mid_body, grid=(_G,),
    in_specs=[_half_spec, _deg_spec, _b_spec, _w_spec],
    out_specs=_half_spec, out_shape=_half_sds)

_tc_post = pl.pallas_call(
    _tc_post_body, grid=(_G,),
    in_specs=[_half_spec, _deg_spec, _b_spec],
    out_specs=_row_spec, out_shape=jax.ShapeDtypeStruct((NP, D), jnp.float32))


def kernel(x, edge_index, edge_weight, W1, b1, W2, b2):
    pad = EP - E
    src = jnp.pad(edge_index[0], (0, pad)).reshape(EP // _CH, _CH)
    dst = jnp.pad(edge_index[1], (0, pad)).reshape(EP // _CH, _CH)
    ew2 = jnp.pad(edge_weight, (0, pad)).reshape(EP // _CH, _CH)
    b1r = b1.reshape(1, D)
    b2r = b2.reshape(1, D)
    xp = jnp.pad(x, ((0, NP - N), (0, 0)))

    pdeg = _get_sc_deg()(dst, ew2)
    pdegT = pdeg.T

    sc_edge = _get_sc_edge()
    hp1 = _tc_pre(xp, W1, pdegT)
    acc1 = sc_edge(hp1, src, dst, ew2)
    hp2 = _tc_mid(acc1, pdegT, b1r, W2)
    acc2 = sc_edge(hp2, src, dst, ew2)
    return _tc_post(acc2, pdegT, b2r)[:N]


# speedup vs baseline: 1.8084x; 1.1370x over previous
import functools

import jax
import jax.numpy as jnp
from jax import lax
from jax.experimental import pallas as pl
from jax.experimental.pallas import tpu as pltpu
from jax.experimental.pallas import tpu_sc as plsc

N = 10000
NP = 10240
E = 320000
EP = 327680
D = 128
DH = 64
NC = 2
NS = 16
L = 16

@functools.lru_cache(maxsize=None)
def _mesh():
    return plsc.VectorSubcoreMesh(core_axis_name="c", subcore_axis_name="s",
                                  num_cores=NC, num_subcores=NS)

_GDN = lax.GatherDimensionNumbers(
    offset_dims=(), collapsed_slice_dims=(0,), start_index_map=(0,))


def _lane_bcast(vec16, j):
    idx = jnp.full((L, 1), j, jnp.int32)
    return lax.gather(vec16, idx, _GDN, slice_sizes=(1,),
                      mode=lax.GatherScatterMode.PROMISE_IN_BOUNDS)
_RPT = NP // NS

_CHD = 128
_EPTD = EP // (NC * NS)
_NCHD = _EPTD // _CHD


_BURST = 5


def _deg_body(dst_hbm, ew_hbm, pdeg_hbm, degsh, dbg, ebg, zbuf, sem):
    c = lax.axis_index("c")
    s = lax.axis_index("s")

    def _z(i, _):
        zbuf[pl.ds(i * L, L)] = jnp.zeros((L,), jnp.float32)
        return 0
    lax.fori_loop(0, _RPT // L, _z, 0)
    pltpu.sync_copy(zbuf, degsh.at[pl.ds(s * _RPT, _RPT)])

    row0 = (s * NC + c) * _NCHD
    pltpu.sync_copy(dst_hbm.at[pl.ds(row0, _NCHD), :], dbg)
    pltpu.sync_copy(ew_hbm.at[pl.ds(row0, _NCHD), :], ebg)
    plsc.subcore_barrier()

    def _burst(i, _):
        for j in range(_BURST):
            k = i * _BURST + j
            pltpu.async_copy(ebg.at[k], degsh.at[dbg.at[k]], sem, add=True)
        for j in range(_BURST):
            k = i * _BURST + j
            pltpu.make_async_copy(ebg.at[k], degsh.at[dbg.at[k]], sem).wait()
        return 0
    lax.fori_loop(0, _NCHD // _BURST, _burst, 0)

    plsc.subcore_barrier()
    pltpu.sync_copy(degsh.at[pl.ds(s * _RPT, _RPT)],
                    pdeg_hbm.at[c, pl.ds(s * _RPT, _RPT)])


@functools.lru_cache(maxsize=None)
def _get_sc_deg():
    return functools.partial(
        pl.kernel, _deg_body,
        out_type=jax.ShapeDtypeStruct((NC, NP), jnp.float32),
        mesh=_mesh(),
        scratch_types=[
            pltpu.VMEM_SHARED((NP,), jnp.float32),
            pltpu.VMEM((_NCHD, _CHD), jnp.int32),
            pltpu.VMEM((_NCHD, _CHD), jnp.float32),
            pltpu.VMEM((_RPT,), jnp.float32),
            pltpu.SemaphoreType.DMA,
        ],
        compiler_params=pltpu.CompilerParams(use_tc_tiling_on_sc=False),
    )()

_CH = 128
_EPT = EP // NS
_NCHE = _EPT // _CH
_SEG = 40


def _edge_body(hp_hbm, src_hbm, dst_hbm, ew_hbm, acc_hbm,
               table, accsh, sb, db, eb, rows0, rows1,
               gsem0, gsem1, ssem0, ssem1):
    c = lax.axis_index("c")
    s = lax.axis_index("s")
    rbase = s * _RPT

    pltpu.sync_copy(hp_hbm.at[c, pl.ds(rbase, _RPT), :],
                    table.at[pl.ds(rbase, _RPT)])
    pltpu.sync_copy(hp_hbm.at[c, pl.ds(rbase, _RPT), :],
                    accsh.at[pl.ds(rbase, _RPT)])
    plsc.subcore_barrier()

    bufs = ((rows0, gsem0, ssem0), (rows1, gsem1, ssem1))

    def _start_gather(k, b):
        pltpu.async_copy(table.at[sb.at[k]], b[0], b[1])

    def _wait_gather(k, b):
        pltpu.make_async_copy(table.at[sb.at[k]], b[0], b[1]).wait()

    def _scale(k, b):
        rows = b[0]
        nf = DH // L

        def _grp(g, _):
            ew16 = eb[k, pl.ds(g * L, L)]
            for j in range(0, L, 4):
                es = [g * L + j + t for t in range(4)]
                ewbs = [_lane_bcast(ew16, j + t) for t in range(4)]
                vals = [rows[e, pl.ds(f * L, L)] * w
                        for e, w in zip(es, ewbs) for f in range(nf)]
                i = 0
                for e in es:
                    for f in range(nf):
                        rows[e, pl.ds(f * L, L)] = vals[i]
                        i += 1
            return 0
        lax.fori_loop(0, _CH // L, _grp, 0)

    def _start_scatter(k, b):
        pltpu.async_copy(b[0], accsh.at[db.at[k]], b[2], add=True)

    def _wait_scatter(k, b):
        pltpu.make_async_copy(b[0], accsh.at[db.at[k]], b[2]).wait()

    def _seg(gi, _):
        segrow = s * _NCHE + gi * _SEG
        pltpu.sync_copy(src_hbm.at[pl.ds(segrow, _SEG), :], sb)
        pltpu.sync_copy(dst_hbm.at[pl.ds(segrow, _SEG), :], db)
        pltpu.sync_copy(ew_hbm.at[pl.ds(segrow, _SEG), :], eb)
        _start_gather(0, bufs[0])

        def _pair(i, _):
            k0 = 2 * i
            @pl.when(i > 0)
            def _():
                _wait_scatter(k0 - 1, bufs[1])
            _start_gather(k0 + 1, bufs[1])
            _wait_gather(k0, bufs[0])
            _scale(k0, bufs[0])
            _start_scatter(k0, bufs[0])
            @pl.when(i < _SEG // 2 - 1)
            def _():
                _wait_scatter(k0, bufs[0])
                _start_gather(k0 + 2, bufs[0])
            _wait_gather(k0 + 1, bufs[1])
            _scale(k0 + 1, bufs[1])
            _start_scatter(k0 + 1, bufs[1])
            return 0
        lax.fori_loop(0, _SEG // 2, _pair, 0)
        _wait_scatter(_SEG - 2, bufs[0])
        _wait_scatter(_SEG - 1, bufs[1])
        return 0
    lax.fori_loop(0, _NCHE // _SEG, _seg, 0)

    plsc.subcore_barrier()
    pltpu.sync_copy(accsh.at[pl.ds(rbase, _RPT)],
                    acc_hbm.at[c, pl.ds(rbase, _RPT), :])


@functools.lru_cache(maxsize=None)
def _get_sc_edge():
    return functools.partial(
        pl.kernel, _edge_body,
        out_type=jax.ShapeDtypeStruct((NC, NP, DH), jnp.float32),
        mesh=_mesh(),
        scratch_types=[
            pltpu.VMEM_SHARED((NP, DH), jnp.float32),
            pltpu.VMEM_SHARED((NP, DH), jnp.float32),
            pltpu.VMEM((_SEG, _CH), jnp.int32),
            pltpu.VMEM((_SEG, _CH), jnp.int32),
            pltpu.VMEM((_SEG, _CH), jnp.float32),
            pltpu.VMEM((_CH, DH), jnp.float32),
            pltpu.VMEM((_CH, DH), jnp.float32),
            pltpu.SemaphoreType.DMA,
            pltpu.SemaphoreType.DMA,
            pltpu.SemaphoreType.DMA,
            pltpu.SemaphoreType.DMA,
        ],
        compiler_params=pltpu.CompilerParams(use_tc_tiling_on_sc=False),
    )()

_R = 1024
_G = NP // _R


def _dinv(pdeg_ref):
    deg = pdeg_ref[:, 0:1] + pdeg_ref[:, 1:2] + 1.0
    return lax.rsqrt(deg)


def _tc_pre_body(x_ref, w_ref, pdeg_ref, o_ref):
    res = (jnp.dot(x_ref[...], w_ref[...],
                   preferred_element_type=jnp.float32) * _dinv(pdeg_ref))
    o_ref[0, :, :] = res[:, :DH]
    o_ref[1, :, :] = res[:, DH:]


def _tc_mid_body(a_ref, pdeg_ref, b_ref, w_ref, o_ref):
    dinv = _dinv(pdeg_ref)
    a = jnp.concatenate([a_ref[0], a_ref[1]], axis=-1)
    h = jnp.maximum(a * dinv + b_ref[...], 0.0)
    res = (jnp.dot(h, w_ref[...], preferred_element_type=jnp.float32) * dinv)
    o_ref[0, :, :] = res[:, :DH]
    o_ref[1, :, :] = res[:, DH:]


def _tc_post_body(a_ref, pdeg_ref, b_ref, o_ref):
    a = jnp.concatenate([a_ref[0], a_ref[1]], axis=-1)
    o_ref[...] = jnp.maximum(a * _dinv(pdeg_ref) + b_ref[...], 0.0)


_row_spec = pl.BlockSpec((_R, D), lambda i: (i, 0))
_half_spec = pl.BlockSpec((NC, _R, DH), lambda i: (0, i, 0))
_deg_spec = pl.BlockSpec((_R, 2), lambda i: (i, 0))
_w_spec = pl.BlockSpec((D, D), lambda i: (0, 0))
_b_spec = pl.BlockSpec((1, D), lambda i: (0, 0))
_half_sds = jax.ShapeDtypeStruct((NC, NP, DH), jnp.float32)

_tc_pre = pl.pallas_call(
    _tc_pre_body, grid=(_G,),
    in_specs=[_row_spec, _w_spec, _deg_spec],
    out_specs=_half_spec, out_shape=_half_sds)

_tc_mid = pl.pallas_call(
    _tc_mid_body, grid=(_G,),
    in_specs=[_half_spec, _deg_spec, _b_spec, _w_spec],
    out_specs=_half_spec, out_shape=_half_sds)

_tc_post = pl.pallas_call(
    _tc_post_body, grid=(_G,),
    in_specs=[_half_spec, _deg_spec, _b_spec],
    out_specs=_row_spec, out_shape=jax.ShapeDtypeStruct((NP, D), jnp.float32))


def kernel(x, edge_index, edge_weight, W1, b1, W2, b2):
    pad = EP - E
    src = jnp.pad(edge_index[0], (0, pad)).reshape(EP // _CH, _CH)
    dst = jnp.pad(edge_index[1], (0, pad)).reshape(EP // _CH, _CH)
    ew2 = jnp.pad(edge_weight, (0, pad)).reshape(EP // _CH, _CH)
    b1r = b1.reshape(1, D)
    b2r = b2.reshape(1, D)
    xp = jnp.pad(x, ((0, NP - N), (0, 0)))

    pdeg = _get_sc_deg()(dst, ew2)
    pdegT = pdeg.T

    sc_edge = _get_sc_edge()
    hp1 = _tc_pre(xp, W1, pdegT)
    acc1 = sc_edge(hp1, src, dst, ew2)
    hp2 = _tc_mid(acc1, pdegT, b1r, W2)
    acc2 = sc_edge(hp2, src, dst, ew2)
    return _tc_post(acc2, pdegT, b2r)[:N]


# speedup vs baseline: 1.8525x; 1.0244x over previous
import functools

import jax
import jax.numpy as jnp
from jax import lax
from jax.experimental import pallas as pl
from jax.experimental.pallas import tpu as pltpu
from jax.experimental.pallas import tpu_sc as plsc

N = 10000
NP = 10240
E = 320000
EP = 327680
D = 128
DH = 64
NC = 2
NS = 16
L = 16

@functools.lru_cache(maxsize=None)
def _mesh():
    return plsc.VectorSubcoreMesh(core_axis_name="c", subcore_axis_name="s",
                                  num_cores=NC, num_subcores=NS)

_GDN = lax.GatherDimensionNumbers(
    offset_dims=(), collapsed_slice_dims=(0,), start_index_map=(0,))


def _lane_bcast(vec16, j):
    idx = jnp.full((L, 1), j, jnp.int32)
    return lax.gather(vec16, idx, _GDN, slice_sizes=(1,),
                      mode=lax.GatherScatterMode.PROMISE_IN_BOUNDS)
_RPT = NP // NS

_CHD = 128
_EPTD = EP // (NC * NS)
_NCHD = _EPTD // _CHD


_BURST = 5


def _deg_body(dst_hbm, ew_hbm, pdeg_hbm, degsh, dbg, ebg, zbuf, sem):
    c = lax.axis_index("c")
    s = lax.axis_index("s")

    def _z(i, _):
        zbuf[pl.ds(i * L, L)] = jnp.zeros((L,), jnp.float32)
        return 0
    lax.fori_loop(0, _RPT // L, _z, 0)
    pltpu.sync_copy(zbuf, degsh.at[pl.ds(s * _RPT, _RPT)])

    row0 = (s * NC + c) * _NCHD
    pltpu.sync_copy(dst_hbm.at[pl.ds(row0, _NCHD), :], dbg)
    pltpu.sync_copy(ew_hbm.at[pl.ds(row0, _NCHD), :], ebg)
    plsc.subcore_barrier()

    def _burst(i, _):
        for j in range(_BURST):
            k = i * _BURST + j
            pltpu.async_copy(ebg.at[k], degsh.at[dbg.at[k]], sem, add=True)
        for j in range(_BURST):
            k = i * _BURST + j
            pltpu.make_async_copy(ebg.at[k], degsh.at[dbg.at[k]], sem).wait()
        return 0
    lax.fori_loop(0, _NCHD // _BURST, _burst, 0)

    plsc.subcore_barrier()
    pltpu.sync_copy(degsh.at[pl.ds(s * _RPT, _RPT)],
                    pdeg_hbm.at[c, pl.ds(s * _RPT, _RPT)])


@functools.lru_cache(maxsize=None)
def _get_sc_deg():
    return functools.partial(
        pl.kernel, _deg_body,
        out_type=jax.ShapeDtypeStruct((NC, NP), jnp.float32),
        mesh=_mesh(),
        scratch_types=[
            pltpu.VMEM_SHARED((NP,), jnp.float32),
            pltpu.VMEM((_NCHD, _CHD), jnp.int32),
            pltpu.VMEM((_NCHD, _CHD), jnp.float32),
            pltpu.VMEM((_RPT,), jnp.float32),
            pltpu.SemaphoreType.DMA,
        ],
        compiler_params=pltpu.CompilerParams(use_tc_tiling_on_sc=False),
    )()

_CH = 128
_EPT = EP // NS
_NCHE = _EPT // _CH
_SEG = 40


def _edge_body(hp_hbm, src_hbm, dst_hbm, ew_hbm, acc_hbm,
               table, accsh, sb, db, eb, rows0, rows1,
               gsem0, gsem1, ssem0, ssem1):
    c = lax.axis_index("c")
    s = lax.axis_index("s")
    rbase = s * _RPT

    pltpu.sync_copy(hp_hbm.at[c, pl.ds(rbase, _RPT), :],
                    table.at[pl.ds(rbase, _RPT)])
    pltpu.sync_copy(hp_hbm.at[c, pl.ds(rbase, _RPT), :],
                    accsh.at[pl.ds(rbase, _RPT)])
    plsc.subcore_barrier()

    bufs = ((rows0, gsem0, ssem0), (rows1, gsem1, ssem1))

    def _start_gather(k, b):
        pltpu.async_copy(table.at[sb.at[k]], b[0], b[1])

    def _wait_gather(k, b):
        pltpu.make_async_copy(table.at[sb.at[k]], b[0], b[1]).wait()

    def _scale(k, b):
        rows = b[0]
        nf = DH // L

        def _grp(g, _):
            ew16 = eb[k, pl.ds(g * L, L)]
            for j in range(0, L, 8):
                es = [g * L + j + t for t in range(8)]
                ewbs = [_lane_bcast(ew16, j + t) for t in range(8)]
                vals = [rows[e, pl.ds(f * L, L)] * w
                        for e, w in zip(es, ewbs) for f in range(nf)]
                i = 0
                for e in es:
                    for f in range(nf):
                        rows[e, pl.ds(f * L, L)] = vals[i]
                        i += 1
            return 0
        lax.fori_loop(0, _CH // L, _grp, 0)

    def _start_scatter(k, b):
        pltpu.async_copy(b[0], accsh.at[db.at[k]], b[2], add=True)

    def _wait_scatter(k, b):
        pltpu.make_async_copy(b[0], accsh.at[db.at[k]], b[2]).wait()

    def _seg(gi, _):
        segrow = s * _NCHE + gi * _SEG
        pltpu.sync_copy(src_hbm.at[pl.ds(segrow, _SEG), :], sb)
        pltpu.sync_copy(dst_hbm.at[pl.ds(segrow, _SEG), :], db)
        pltpu.sync_copy(ew_hbm.at[pl.ds(segrow, _SEG), :], eb)
        _start_gather(0, bufs[0])

        def _pair(i, _):
            k0 = 2 * i
            @pl.when(i > 0)
            def _():
                _wait_scatter(k0 - 1, bufs[1])
            _start_gather(k0 + 1, bufs[1])
            _wait_gather(k0, bufs[0])
            _scale(k0, bufs[0])
            _start_scatter(k0, bufs[0])
            @pl.when(i < _SEG // 2 - 1)
            def _():
                _wait_scatter(k0, bufs[0])
                _start_gather(k0 + 2, bufs[0])
            _wait_gather(k0 + 1, bufs[1])
            _scale(k0 + 1, bufs[1])
            _start_scatter(k0 + 1, bufs[1])
            return 0
        lax.fori_loop(0, _SEG // 2, _pair, 0)
        _wait_scatter(_SEG - 2, bufs[0])
        _wait_scatter(_SEG - 1, bufs[1])
        return 0
    lax.fori_loop(0, _NCHE // _SEG, _seg, 0)

    plsc.subcore_barrier()
    pltpu.sync_copy(accsh.at[pl.ds(rbase, _RPT)],
                    acc_hbm.at[c, pl.ds(rbase, _RPT), :])


@functools.lru_cache(maxsize=None)
def _get_sc_edge():
    return functools.partial(
        pl.kernel, _edge_body,
        out_type=jax.ShapeDtypeStruct((NC, NP, DH), jnp.float32),
        mesh=_mesh(),
        scratch_types=[
            pltpu.VMEM_SHARED((NP, DH), jnp.float32),
            pltpu.VMEM_SHARED((NP, DH), jnp.float32),
            pltpu.VMEM((_SEG, _CH), jnp.int32),
            pltpu.VMEM((_SEG, _CH), jnp.int32),
            pltpu.VMEM((_SEG, _CH), jnp.float32),
            pltpu.VMEM((_CH, DH), jnp.float32),
            pltpu.VMEM((_CH, DH), jnp.float32),
            pltpu.SemaphoreType.DMA,
            pltpu.SemaphoreType.DMA,
            pltpu.SemaphoreType.DMA,
            pltpu.SemaphoreType.DMA,
        ],
        compiler_params=pltpu.CompilerParams(use_tc_tiling_on_sc=False),
    )()

_R = 1024
_G = NP // _R


def _dinv(pdeg_ref):
    deg = pdeg_ref[:, 0:1] + pdeg_ref[:, 1:2] + 1.0
    return lax.rsqrt(deg)


def _tc_pre_body(x_ref, w_ref, pdeg_ref, o_ref):
    res = (jnp.dot(x_ref[...], w_ref[...],
                   preferred_element_type=jnp.float32) * _dinv(pdeg_ref))
    o_ref[0, :, :] = res[:, :DH]
    o_ref[1, :, :] = res[:, DH:]


def _tc_mid_body(a_ref, pdeg_ref, b_ref, w_ref, o_ref):
    dinv = _dinv(pdeg_ref)
    a = jnp.concatenate([a_ref[0], a_ref[1]], axis=-1)
    h = jnp.maximum(a * dinv + b_ref[...], 0.0)
    res = (jnp.dot(h, w_ref[...], preferred_element_type=jnp.float32) * dinv)
    o_ref[0, :, :] = res[:, :DH]
    o_ref[1, :, :] = res[:, DH:]


def _tc_post_body(a_ref, pdeg_ref, b_ref, o_ref):
    a = jnp.concatenate([a_ref[0], a_ref[1]], axis=-1)
    o_ref[...] = jnp.maximum(a * _dinv(pdeg_ref) + b_ref[...], 0.0)


_row_spec = pl.BlockSpec((_R, D), lambda i: (i, 0))
_half_spec = pl.BlockSpec((NC, _R, DH), lambda i: (0, i, 0))
_deg_spec = pl.BlockSpec((_R, 2), lambda i: (i, 0))
_w_spec = pl.BlockSpec((D, D), lambda i: (0, 0))
_b_spec = pl.BlockSpec((1, D), lambda i: (0, 0))
_half_sds = jax.ShapeDtypeStruct((NC, NP, DH), jnp.float32)

_tc_pre = pl.pallas_call(
    _tc_pre_body, grid=(_G,),
    in_specs=[_row_spec, _w_spec, _deg_spec],
    out_specs=_half_spec, out_shape=_half_sds)

_tc_mid = pl.pallas_call(
    _tc_mid_body, grid=(_G,),
    in_specs=[_half_spec, _deg_spec, _b_spec, _w_spec],
    out_specs=_half_spec, out_shape=_half_sds)

_tc_post = pl.pallas_call(
    _tc_post_body, grid=(_G,),
    in_specs=[_half_spec, _deg_spec, _b_spec],
    out_specs=_row_spec, out_shape=jax.ShapeDtypeStruct((NP, D), jnp.float32))


def kernel(x, edge_index, edge_weight, W1, b1, W2, b2):
    pad = EP - E
    src = jnp.pad(edge_index[0], (0, pad)).reshape(EP // _CH, _CH)
    dst = jnp.pad(edge_index[1], (0, pad)).reshape(EP // _CH, _CH)
    ew2 = jnp.pad(edge_weight, (0, pad)).reshape(EP // _CH, _CH)
    b1r = b1.reshape(1, D)
    b2r = b2.reshape(1, D)
    xp = jnp.pad(x, ((0, NP - N), (0, 0)))

    pdeg = _get_sc_deg()(dst, ew2)
    pdegT = pdeg.T

    sc_edge = _get_sc_edge()
    hp1 = _tc_pre(xp, W1, pdegT)
    acc1 = sc_edge(hp1, src, dst, ew2)
    hp2 = _tc_mid(acc1, pdegT, b1r, W2)
    acc2 = sc_edge(hp2, src, dst, ew2)
    return _tc_post(acc2, pdegT, b2r)[:N]


# speedup vs baseline: 2.1362x; 1.1532x over previous
import functools

import jax
import jax.numpy as jnp
from jax import lax
from jax.experimental import pallas as pl
from jax.experimental.pallas import tpu as pltpu
from jax.experimental.pallas import tpu_sc as plsc

N = 10000
NP = 10240
E = 320000
EP = 327680
D = 128
DH = 64
NC = 2
NS = 16
L = 16

@functools.lru_cache(maxsize=None)
def _mesh():
    return plsc.VectorSubcoreMesh(core_axis_name="c", subcore_axis_name="s",
                                  num_cores=NC, num_subcores=NS)

_GDN = lax.GatherDimensionNumbers(
    offset_dims=(), collapsed_slice_dims=(0,), start_index_map=(0,))


def _lane_bcast(vec16, j):
    idx = jnp.full((L, 1), j, jnp.int32)
    return lax.gather(vec16, idx, _GDN, slice_sizes=(1,),
                      mode=lax.GatherScatterMode.PROMISE_IN_BOUNDS)
_RPT = NP // NS

_CHD = 128
_EPTD = EP // (NC * NS)
_NCHD = _EPTD // _CHD


_BURST = 5


def _deg_body(dst_hbm, ew_hbm, pdeg_hbm, degsh, dbg, ebg, zbuf, sem):
    c = lax.axis_index("c")
    s = lax.axis_index("s")

    def _z(i, _):
        zbuf[pl.ds(i * L, L)] = jnp.zeros((L,), jnp.float32)
        return 0
    lax.fori_loop(0, _RPT // L, _z, 0)
    pltpu.sync_copy(zbuf, degsh.at[pl.ds(s * _RPT, _RPT)])

    row0 = (s * NC + c) * _NCHD
    pltpu.sync_copy(dst_hbm.at[pl.ds(row0, _NCHD), :], dbg)
    pltpu.sync_copy(ew_hbm.at[pl.ds(row0, _NCHD), :], ebg)
    plsc.subcore_barrier()

    def _burst(i, _):
        for j in range(_BURST):
            k = i * _BURST + j
            pltpu.async_copy(ebg.at[k], degsh.at[dbg.at[k]], sem, add=True)
        for j in range(_BURST):
            k = i * _BURST + j
            pltpu.make_async_copy(ebg.at[k], degsh.at[dbg.at[k]], sem).wait()
        return 0
    lax.fori_loop(0, _NCHD // _BURST, _burst, 0)

    plsc.subcore_barrier()
    pltpu.sync_copy(degsh.at[pl.ds(s * _RPT, _RPT)],
                    pdeg_hbm.at[c, pl.ds(s * _RPT, _RPT)])


@functools.lru_cache(maxsize=None)
def _get_sc_deg():
    return functools.partial(
        pl.kernel, _deg_body,
        out_type=jax.ShapeDtypeStruct((NC, NP), jnp.float32),
        mesh=_mesh(),
        scratch_types=[
            pltpu.VMEM_SHARED((NP,), jnp.float32),
            pltpu.VMEM((_NCHD, _CHD), jnp.int32),
            pltpu.VMEM((_NCHD, _CHD), jnp.float32),
            pltpu.VMEM((_RPT,), jnp.float32),
            pltpu.SemaphoreType.DMA,
        ],
        compiler_params=pltpu.CompilerParams(use_tc_tiling_on_sc=False),
    )()

_CH = 128
_EPT = EP // NS
_NCHE = _EPT // _CH
_SEG = 32


def _edge_body(hp_hbm, src_hbm, dst_hbm, ew_hbm, acc_hbm,
               table, accsh, sb, db, eb, rows0, rows1, rows2, rows3,
               gsem0, gsem1, gsem2, gsem3, ssem0, ssem1, ssem2, ssem3):
    c = lax.axis_index("c")
    s = lax.axis_index("s")
    rbase = s * _RPT

    pltpu.sync_copy(hp_hbm.at[c, pl.ds(rbase, _RPT), :],
                    table.at[pl.ds(rbase, _RPT)])
    pltpu.sync_copy(hp_hbm.at[c, pl.ds(rbase, _RPT), :],
                    accsh.at[pl.ds(rbase, _RPT)])
    plsc.subcore_barrier()

    bufs = ((rows0, gsem0, ssem0), (rows1, gsem1, ssem1),
            (rows2, gsem2, ssem2), (rows3, gsem3, ssem3))

    def _start_gather(k, b):
        pltpu.async_copy(table.at[sb.at[k]], b[0], b[1])

    def _wait_gather(k, b):
        pltpu.make_async_copy(table.at[sb.at[k]], b[0], b[1]).wait()

    def _scale(k, b):
        rows = b[0]
        nf = DH // L

        def _grp(g, _):
            ew16 = eb[k, pl.ds(g * L, L)]
            for j in range(0, L, 8):
                es = [g * L + j + t for t in range(8)]
                ewbs = [_lane_bcast(ew16, j + t) for t in range(8)]
                vals = [rows[e, pl.ds(f * L, L)] * w
                        for e, w in zip(es, ewbs) for f in range(nf)]
                i = 0
                for e in es:
                    for f in range(nf):
                        rows[e, pl.ds(f * L, L)] = vals[i]
                        i += 1
            return 0
        lax.fori_loop(0, _CH // L, _grp, 0)

    def _start_scatter(k, b):
        pltpu.async_copy(b[0], accsh.at[db.at[k]], b[2], add=True)

    def _wait_scatter(k, b):
        pltpu.make_async_copy(b[0], accsh.at[db.at[k]], b[2]).wait()

    def _seg(gi, _):
        segrow = s * _NCHE + gi * _SEG
        pltpu.sync_copy(src_hbm.at[pl.ds(segrow, _SEG), :], sb)
        pltpu.sync_copy(dst_hbm.at[pl.ds(segrow, _SEG), :], db)
        pltpu.sync_copy(ew_hbm.at[pl.ds(segrow, _SEG), :], eb)
        _start_gather(0, bufs[0])
        _start_gather(1, bufs[1])

        def _quad(i, _):
            for j in range(4):
                k = 4 * i + j
                b = bufs[j]
                _wait_gather(k, b)
                _scale(k, b)
                _start_scatter(k, b)
                kk = k + 2
                bb = bufs[(j + 2) % 4]

                @pl.when(kk < _SEG)
                def _():
                    @pl.when(k >= 2)
                    def _():
                        _wait_scatter(k - 2, bb)
                    _start_gather(kk, bb)
            return 0
        lax.fori_loop(0, _SEG // 4, _quad, 0)
        for j in range(4):
            _wait_scatter(_SEG - 4 + j, bufs[j])
        return 0
    lax.fori_loop(0, _NCHE // _SEG, _seg, 0)

    plsc.subcore_barrier()
    pltpu.sync_copy(accsh.at[pl.ds(rbase, _RPT)],
                    acc_hbm.at[c, pl.ds(rbase, _RPT), :])


@functools.lru_cache(maxsize=None)
def _get_sc_edge():
    return functools.partial(
        pl.kernel, _edge_body,
        out_type=jax.ShapeDtypeStruct((NC, NP, DH), jnp.float32),
        mesh=_mesh(),
        scratch_types=[
            pltpu.VMEM_SHARED((NP, DH), jnp.float32),
            pltpu.VMEM_SHARED((NP, DH), jnp.float32),
            pltpu.VMEM((_SEG, _CH), jnp.int32),
            pltpu.VMEM((_SEG, _CH), jnp.int32),
            pltpu.VMEM((_SEG, _CH), jnp.float32),
            pltpu.VMEM((_CH, DH), jnp.float32),
            pltpu.VMEM((_CH, DH), jnp.float32),
            pltpu.VMEM((_CH, DH), jnp.float32),
            pltpu.VMEM((_CH, DH), jnp.float32),
            pltpu.SemaphoreType.DMA,
            pltpu.SemaphoreType.DMA,
            pltpu.SemaphoreType.DMA,
            pltpu.SemaphoreType.DMA,
            pltpu.SemaphoreType.DMA,
            pltpu.SemaphoreType.DMA,
            pltpu.SemaphoreType.DMA,
            pltpu.SemaphoreType.DMA,
        ],
        compiler_params=pltpu.CompilerParams(use_tc_tiling_on_sc=False),
    )()

_R = 1024
_G = NP // _R


def _dinv(pdeg_ref):
    deg = pdeg_ref[:, 0:1] + pdeg_ref[:, 1:2] + 1.0
    return lax.rsqrt(deg)


def _tc_pre_body(x_ref, w_ref, pdeg_ref, o_ref):
    res = (jnp.dot(x_ref[...], w_ref[...],
                   preferred_element_type=jnp.float32) * _dinv(pdeg_ref))
    o_ref[0, :, :] = res[:, :DH]
    o_ref[1, :, :] = res[:, DH:]


def _tc_mid_body(a_ref, pdeg_ref, b_ref, w_ref, o_ref):
    dinv = _dinv(pdeg_ref)
    a = jnp.concatenate([a_ref[0], a_ref[1]], axis=-1)
    h = jnp.maximum(a * dinv + b_ref[...], 0.0)
    res = (jnp.dot(h, w_ref[...], preferred_element_type=jnp.float32) * dinv)
    o_ref[0, :, :] = res[:, :DH]
    o_ref[1, :, :] = res[:, DH:]


def _tc_post_body(a_ref, pdeg_ref, b_ref, o_ref):
    a = jnp.concatenate([a_ref[0], a_ref[1]], axis=-1)
    o_ref[...] = jnp.maximum(a * _dinv(pdeg_ref) + b_ref[...], 0.0)


_row_spec = pl.BlockSpec((_R, D), lambda i: (i, 0))
_half_spec = pl.BlockSpec((NC, _R, DH), lambda i: (0, i, 0))
_deg_spec = pl.BlockSpec((_R, 2), lambda i: (i, 0))
_w_spec = pl.BlockSpec((D, D), lambda i: (0, 0))
_b_spec = pl.BlockSpec((1, D), lambda i: (0, 0))
_half_sds = jax.ShapeDtypeStruct((NC, NP, DH), jnp.float32)

_tc_pre = pl.pallas_call(
    _tc_pre_body, grid=(_G,),
    in_specs=[_row_spec, _w_spec, _deg_spec],
    out_specs=_half_spec, out_shape=_half_sds)

_tc_mid = pl.pallas_call(
    _tc_mid_body, grid=(_G,),
    in_specs=[_half_spec, _deg_spec, _b_spec, _w_spec],
    out_specs=_half_spec, out_shape=_half_sds)

_tc_post = pl.pallas_call(
    _tc_post_body, grid=(_G,),
    in_specs=[_half_spec, _deg_spec, _b_spec],
    out_specs=_row_spec, out_shape=jax.ShapeDtypeStruct((NP, D), jnp.float32))


def kernel(x, edge_index, edge_weight, W1, b1, W2, b2):
    pad = EP - E
    src = jnp.pad(edge_index[0], (0, pad)).reshape(EP // _CH, _CH)
    dst = jnp.pad(edge_index[1], (0, pad)).reshape(EP // _CH, _CH)
    ew2 = jnp.pad(edge_weight, (0, pad)).reshape(EP // _CH, _CH)
    b1r = b1.reshape(1, D)
    b2r = b2.reshape(1, D)
    xp = jnp.pad(x, ((0, NP - N), (0, 0)))

    pdeg = _get_sc_deg()(dst, ew2)
    pdegT = pdeg.T

    sc_edge = _get_sc_edge()
    hp1 = _tc_pre(xp, W1, pdegT)
    acc1 = sc_edge(hp1, src, dst, ew2)
    hp2 = _tc_mid(acc1, pdegT, b1r, W2)
    acc2 = sc_edge(hp2, src, dst, ew2)
    return _tc_post(acc2, pdegT, b2r)[:N]
